# Initial kernel scaffold; baseline (speedup 1.0000x reference)
import jax
import jax.numpy as jnp
from jax.experimental import pallas as pl


def kernel(x, w1m, b1r, w2t, b2r, fw1m, fb1r, fw2m, fb2r):
    raise NotImplementedError("write your pallas kernel here")


# speedup vs baseline: 13.7489x; 13.7489x over previous
import numpy as np

import jax
import jax.numpy as jnp
from jax.experimental import pallas as pl
from jax.experimental.pallas import tpu as pltpu

_OH1 = 31
_OH2 = 15
_C1 = 16
_C2 = 32
_CIN = 3
_LIN = 64 * _CIN
_N1 = _OH1 * _C1
_N2 = _OH2 * _C2


def _sel1():
    s = np.zeros((64, _OH1, 3), np.float32)
    for j in range(_OH1):
        for kj in range(3):
            s[2 * j + kj, j, kj] = 1.0
    return s


def _sel2():
    s = np.zeros((_OH1, _OH2, 3), np.float32)
    for oj in range(_OH2):
        for dj in range(3):
            s[2 * oj + dj, oj, dj] = 1.0
    return s


def _fused_kernel(xp_ref, w1_ref, b1_ref, w2_ref, b2_ref, fw1_ref, fb1_ref,
                  fw2_ref, fb2_ref, o_ref):
    bs = xp_ref.shape[2]
    xp = xp_ref[...]

    def rows(p, lo, hi):
        return xp[p, lo:hi].reshape((hi - lo) * bs, _LIN)

    dot = lambda a, b: jnp.dot(a, b, preferred_element_type=jnp.float32)

    a1e = (dot(rows(0, 0, 16), w1_ref[0]) +
           dot(rows(1, 0, 16), w1_ref[1]) +
           dot(rows(2, 0, 16), w1_ref[2]))
    a1o = (dot(rows(2, 0, 15), w1_ref[0]) +
           dot(rows(3, 0, 15), w1_ref[1]) +
           dot(rows(0, 1, 16), w1_ref[2]))
    a1e = jnp.maximum(a1e + b1_ref[...], 0.0)
    a1o = jnp.maximum(a1o + b1_ref[...], 0.0)

    e3 = a1e.reshape(16, bs, _N1)
    acc = (dot(e3[0:15].reshape(15 * bs, _N1), w2_ref[0]) +
           dot(a1o, w2_ref[1]) +
           dot(e3[1:16].reshape(15 * bs, _N1), w2_ref[2]))
    a2 = jnp.maximum(acc + b2_ref[...], 0.0)
    a23 = a2.reshape(15, bs, _N2)

    h = dot(a23[0], fw1_ref[0])
    for oi in range(1, _OH2):
        h = h + dot(a23[oi], fw1_ref[oi])
    h = jnp.maximum(h + fb1_ref[...], 0.0)
    o_ref[...] = dot(h, fw2_ref[...]) + fb2_ref[...]


def kernel(x, w1m, b1r, w2t, b2r, fw1m, fb1r, fw2m, fb2r):
    B = x.shape[0]
    A = fw2m.shape[1]
    bs = next(b for b in (64, 32, 16, 8, 4, 2, 1) if B % b == 0)

    xp = (x.transpose(0, 2, 3, 1)
            .reshape(B, 16, 4, _LIN)
            .transpose(2, 1, 0, 3))

    w1r = w1m.reshape(3, 3, _CIN, _C1)
    w1b = jnp.einsum('wjk,ikcd->iwcjd', _sel1(), w1r).reshape(3, _LIN, _N1)
    w2b = jnp.einsum('jod,idmn->ijmon', _sel2(), w2t).reshape(3, _N1, _N2)
    b1t = jnp.tile(b1r, (1, _OH1))
    b2t = jnp.tile(b2r, (1, _OH2))
    fw1r = fw1m.reshape(_OH2, _N2, 256)

    return pl.pallas_call(
        _fused_kernel,
        out_shape=jax.ShapeDtypeStruct((B, A), jnp.float32),
        grid=(B // bs,),
        in_specs=[
            pl.BlockSpec((4, 16, bs, _LIN), lambda i: (0, 0, i, 0)),
            pl.BlockSpec((3, _LIN, _N1), lambda i: (0, 0, 0)),
            pl.BlockSpec((1, _N1), lambda i: (0, 0)),
            pl.BlockSpec((3, _N1, _N2), lambda i: (0, 0, 0)),
            pl.BlockSpec((1, _N2), lambda i: (0, 0)),
            pl.BlockSpec((_OH2, _N2, 256), lambda i: (0, 0, 0)),
            pl.BlockSpec((1, 256), lambda i: (0, 0)),
            pl.BlockSpec((256, A), lambda i: (0, 0)),
            pl.BlockSpec((1, A), lambda i: (0, 0)),
        ],
        out_specs=pl.BlockSpec((bs, A), lambda i: (i, 0)),
        compiler_params=pltpu.CompilerParams(
            dimension_semantics=("parallel",),
            vmem_limit_bytes=56 * 1024 * 1024),
    )(xp, w1b, b1t, w2b, b2t, fw1r, fb1r, fw2m, fb2r)
